# baseline (speedup 1.0000x reference)
import functools

import jax
import jax.numpy as jnp
from jax import lax
from jax.experimental import pallas as pl
from jax.experimental.pallas import tpu as pltpu
from jax.experimental.pallas import tpu_sc as plsc

EDIM_ = 32
D_ = 2 * EDIM_
W_ = 2 * D_
B_ = 16384
NC_, NS_ = 2, 16
NW_ = NC_ * NS_
BPW_ = B_ // NW_
CH_ = 128
NCH_ = BPW_ // CH_


def _sc_gather(ut2, it2, user_ids, item_ids):
    mesh = plsc.VectorSubcoreMesh(core_axis_name="c", subcore_axis_name="s")

    @functools.partial(
        pl.kernel,
        mesh=mesh,
        out_type=[
            jax.ShapeDtypeStruct((B_, W_), jnp.float32),
            jax.ShapeDtypeStruct((B_, W_), jnp.float32),
        ],
        scratch_types=[
            pltpu.VMEM((BPW_,), jnp.int32),
            pltpu.VMEM((BPW_,), jnp.int32),
            pltpu.VMEM((CH_, W_), jnp.float32),
            pltpu.VMEM((CH_, W_), jnp.float32),
            pltpu.SemaphoreType.DMA,
            pltpu.SemaphoreType.DMA,
        ],
    )
    def gather_kernel(ut_hbm, it_hbm, uid_hbm, iid_hbm, ue_hbm, ie_hbm,
                      uidx_v, iidx_v, buf0, buf1, sem0, sem1):
        wid = lax.axis_index("s") * NC_ + lax.axis_index("c")
        base = wid * BPW_
        pltpu.sync_copy(uid_hbm.at[pl.ds(base, BPW_)], uidx_v)
        pltpu.sync_copy(iid_hbm.at[pl.ds(base, BPW_)], iidx_v)
        for idxv in (uidx_v, iidx_v):
            for i in range(BPW_ // 16):
                sl = pl.ds(i * 16, 16)
                idxv[sl] = lax.shift_right_logical(idxv[sl], 1)

        items = [(ut_hbm, uidx_v, c, ue_hbm) for c in range(NCH_)] + \
                [(it_hbm, iidx_v, c, it_out) for c, it_out in
                 [(c, ie_hbm) for c in range(NCH_)]]
        bufs = (buf0, buf1)
        sems = (sem0, sem1)

        def issue(k):
            tab, idxv, c, _ = items[k]
            return pltpu.async_copy(
                tab.at[idxv.at[pl.ds(c * CH_, CH_)]], bufs[k % 2], sems[k % 2])

        issue(0)
        issue(1)
        for k in range(len(items)):
            tab, idxv, c, out = items[k]
            pltpu.make_async_copy(
                tab.at[idxv.at[pl.ds(c * CH_, CH_)]], bufs[k % 2],
                sems[k % 2]).wait()
            pltpu.sync_copy(bufs[k % 2], out.at[pl.ds(base + c * CH_, CH_)])
            if k + 2 < len(items):
                issue(k + 2)

    return gather_kernel(ut2, it2, user_ids, item_ids)


def _tc_head_body(ue_ref, ie_ref, up_ref, ip_ref, w1_ref, b1_ref,
                  w2_ref, b2_ref, w3_ref, b3_ref, o_ref):
    upar = up_ref[...].astype(jnp.float32)
    ipar = ip_ref[...].astype(jnp.float32)
    ue2 = ue_ref[...]
    ie2 = ie_ref[...]
    ue = ue2[:, :D_] * (1.0 - upar) + ue2[:, D_:] * upar
    ie = ie2[:, :D_] * (1.0 - ipar) + ie2[:, D_:] * ipar
    gmf = ue[:, :EDIM_] * ie[:, :EDIM_]
    x = jnp.concatenate([ue[:, EDIM_:], ie[:, EDIM_:]], axis=1)
    h1 = lax.dot_general(x, w1_ref[...], (((1,), (1,)), ((), ())),
                         preferred_element_type=jnp.float32)
    h1 = jnp.maximum(h1 + b1_ref[...], 0.0)
    h2 = lax.dot_general(h1, w2_ref[...], (((1,), (1,)), ((), ())),
                         preferred_element_type=jnp.float32)
    h2 = jnp.maximum(h2 + b2_ref[...], 0.0)
    z = jnp.concatenate([gmf, h2], axis=1)
    o = lax.dot_general(z, w3_ref[...], (((1,), (1,)), ((), ())),
                        preferred_element_type=jnp.float32)
    o_ref[...] = o + b3_ref[0]


BT_ = 2048


def _tc_head(ue2, ie2, uid, iid, W1, b1, W2, b2, W3, b3):
    full = lambda shape: pl.BlockSpec(shape, lambda i: (0, 0))
    out = pl.pallas_call(
        _tc_head_body,
        grid=(B_ // BT_,),
        in_specs=[
            pl.BlockSpec((BT_, W_), lambda i: (i, 0)),
            pl.BlockSpec((BT_, W_), lambda i: (i, 0)),
            pl.BlockSpec((BT_, 1), lambda i: (i, 0)),
            pl.BlockSpec((BT_, 1), lambda i: (i, 0)),
            full((EDIM_, D_)),
            full((1, EDIM_)),
            full((EDIM_ // 2, EDIM_)),
            full((1, EDIM_ // 2)),
            full((8, EDIM_ + EDIM_ // 2)),
            pl.BlockSpec(memory_space=pltpu.MemorySpace.SMEM),
        ],
        out_specs=pl.BlockSpec((BT_, 8), lambda i: (i, 0)),
        out_shape=jax.ShapeDtypeStruct((B_, 8), jnp.float32),
    )(ue2, ie2, (uid & 1).reshape(B_, 1), (iid & 1).reshape(B_, 1),
      W1, b1.reshape(1, EDIM_), W2, b2.reshape(1, EDIM_ // 2),
      jnp.broadcast_to(W3, (8, EDIM_ + EDIM_ // 2)), b3)
    return out[:, 0]


def kernel(user_ids, item_ids, user_table, item_table, W1, b1, W2, b2, W3, b3):
    uid = user_ids.astype(jnp.int32)
    iid = item_ids.astype(jnp.int32)
    ut2 = user_table.reshape(-1, W_)
    it2 = item_table.reshape(-1, W_)
    ue2, ie2 = _sc_gather(ut2, it2, uid, iid)
    return _tc_head(ue2, ie2, uid, iid, W1, b1, W2, b2, W3, b3)


# speedup vs baseline: 1.0072x; 1.0072x over previous
import functools

import jax
import jax.numpy as jnp
from jax import lax
from jax.experimental import pallas as pl
from jax.experimental.pallas import tpu as pltpu
from jax.experimental.pallas import tpu_sc as plsc

EDIM_ = 32
D_ = 2 * EDIM_
B_ = 16384
NC_, NS_ = 2, 16
NW_ = NC_ * NS_
BPW_ = B_ // NW_


def _sc_gather(user_table, item_table, user_ids, item_ids):
    mesh = plsc.VectorSubcoreMesh(core_axis_name="c", subcore_axis_name="s")

    @functools.partial(
        pl.kernel,
        mesh=mesh,
        compiler_params=pltpu.CompilerParams(use_tc_tiling_on_sc=False),
        out_type=[
            jax.ShapeDtypeStruct((B_, D_), jnp.float32),
            jax.ShapeDtypeStruct((B_, D_), jnp.float32),
        ],
        scratch_types=[
            pltpu.VMEM((BPW_,), jnp.int32),
            pltpu.VMEM((BPW_,), jnp.int32),
            pltpu.VMEM((BPW_, D_), jnp.float32),
            pltpu.VMEM((BPW_, D_), jnp.float32),
            pltpu.SemaphoreType.DMA,
            pltpu.SemaphoreType.DMA,
        ],
    )
    def gather_kernel(ut_hbm, it_hbm, uid_hbm, iid_hbm, ue_hbm, ie_hbm,
                      uidx_v, iidx_v, ur_v, ir_v, sem_u, sem_i):
        wid = lax.axis_index("s") * NC_ + lax.axis_index("c")
        base = wid * BPW_
        pltpu.sync_copy(uid_hbm.at[pl.ds(base, BPW_)], uidx_v)
        pltpu.sync_copy(iid_hbm.at[pl.ds(base, BPW_)], iidx_v)
        cu = pltpu.async_copy(ut_hbm.at[uidx_v], ur_v, sem_u)
        ci = pltpu.async_copy(it_hbm.at[iidx_v], ir_v, sem_i)
        cu.wait()
        pltpu.sync_copy(ur_v, ue_hbm.at[pl.ds(base, BPW_)])
        ci.wait()
        pltpu.sync_copy(ir_v, ie_hbm.at[pl.ds(base, BPW_)])

    return gather_kernel(user_table, item_table, user_ids, item_ids)


def _tc_head_body(ue_ref, ie_ref, w1_ref, b1_ref, w2_ref, b2_ref,
                  w3_ref, b3_ref, o_ref):
    ue = ue_ref[...]
    ie = ie_ref[...]
    gmf = ue[:, :EDIM_] * ie[:, :EDIM_]
    x = jnp.concatenate([ue[:, EDIM_:], ie[:, EDIM_:]], axis=1)
    h1 = lax.dot_general(x, w1_ref[...], (((1,), (1,)), ((), ())),
                         preferred_element_type=jnp.float32)
    h1 = jnp.maximum(h1 + b1_ref[...], 0.0)
    h2 = lax.dot_general(h1, w2_ref[...], (((1,), (1,)), ((), ())),
                         preferred_element_type=jnp.float32)
    h2 = jnp.maximum(h2 + b2_ref[...], 0.0)
    z = jnp.concatenate([gmf, h2], axis=1)
    o = lax.dot_general(z, w3_ref[...], (((1,), (1,)), ((), ())),
                        preferred_element_type=jnp.float32)
    o_ref[...] = o + b3_ref[0]


def _tc_head(ue, ie, W1, b1, W2, b2, W3, b3):
    out = pl.pallas_call(
        _tc_head_body,
        in_specs=[pl.BlockSpec() for _ in range(7)]
        + [pl.BlockSpec(memory_space=pltpu.MemorySpace.SMEM)],
        out_shape=jax.ShapeDtypeStruct((B_, 8), jnp.float32),
    )(ue, ie, W1, b1.reshape(1, EDIM_), W2, b2.reshape(1, EDIM_ // 2),
      jnp.broadcast_to(W3, (8, EDIM_ + EDIM_ // 2)), b3)
    return out[:, 0]


def kernel(user_ids, item_ids, user_table, item_table, W1, b1, W2, b2, W3, b3):
    uid = user_ids.astype(jnp.int32)
    iid = item_ids.astype(jnp.int32)
    ue, ie = _sc_gather(user_table, item_table, uid, iid)
    return _tc_head(ue, ie, W1, b1, W2, b2, W3, b3)
